# baseline (speedup 1.0000x reference)
import functools

import jax
import jax.numpy as jnp
from jax.experimental import pallas as pl
from jax.experimental.pallas import tpu as pltpu

_CHUNKS_PER_BATCH = 8


def _fp_kernel(mask_ref, out_ref):
    n = mask_ref.shape[1]
    iota = jax.lax.broadcasted_iota(jnp.int32, mask_ref.shape, 1)
    cand = jnp.where(mask_ref[...] != 0, n, iota)
    fp = jnp.min(cand, axis=1, keepdims=True)
    out_ref[...] = jnp.broadcast_to(fp, out_ref.shape)


def _pad_copy_kernel(b, n, d, fp_ref, x_ref, start_ref, end_ref, out_ref,
                     zrow_vmem, sem, row_sem):
    cs = n // _CHUNKS_PER_BATCH
    copies = []
    for bi in range(b):
        for c in range(_CHUNKS_PER_BATCH):
            cp = pltpu.make_async_copy(
                x_ref.at[pl.ds((bi * n + c * cs) * d, cs * d)],
                out_ref.at[pl.ds((bi * (n + 2) + 1 + c * cs) * d, cs * d)],
                sem)
            cp.start()
            copies.append(cp)
    zrow_vmem[...] = jnp.zeros_like(zrow_vmem)
    small = []
    for bi in range(b):
        cp = pltpu.make_async_copy(
            start_ref.at[...],
            out_ref.at[pl.ds(bi * (n + 2) * d, d)], row_sem)
        cp.start()
        small.append(cp)
        cp = pltpu.make_async_copy(
            zrow_vmem.at[...],
            out_ref.at[pl.ds((bi * (n + 2) + n + 1) * d, d)], row_sem)
        cp.start()
        small.append(cp)
    for cp in copies:
        cp.wait()
    for cp in small:
        cp.wait()
    endcps = []
    for bi in range(b):
        off = pl.multiple_of((bi * (n + 2) + fp_ref[bi]) * d, d)
        cp = pltpu.make_async_copy(
            end_ref.at[...], out_ref.at[pl.ds(off, d)], row_sem)
        cp.start()
        endcps.append(cp)
    for cp in endcps:
        cp.wait()


def kernel(x, protein_mask, start, end):
    b, n, d = x.shape
    mask_i32 = protein_mask.astype(jnp.int32)
    fp_full = pl.pallas_call(
        _fp_kernel,
        out_shape=jax.ShapeDtypeStruct((b, 128), jnp.int32),
    )(mask_i32)
    fp = fp_full[:, 0]

    out_flat = pl.pallas_call(
        functools.partial(_pad_copy_kernel, b, n, d),
        grid_spec=pltpu.PrefetchScalarGridSpec(
            num_scalar_prefetch=1,
            grid=(),
            in_specs=[
                pl.BlockSpec(memory_space=pltpu.HBM),
                pl.BlockSpec(memory_space=pltpu.VMEM),
                pl.BlockSpec(memory_space=pltpu.VMEM),
            ],
            out_specs=pl.BlockSpec(memory_space=pltpu.HBM),
            scratch_shapes=[
                pltpu.VMEM((d,), jnp.float32),
                pltpu.SemaphoreType.DMA,
                pltpu.SemaphoreType.DMA,
            ],
        ),
        out_shape=jax.ShapeDtypeStruct((b * (n + 2) * d,), jnp.float32),
    )(fp, x.reshape(-1), start, end)
    return out_flat.reshape(b, n + 2, d)


# speedup vs baseline: 17.9740x; 17.9740x over previous
import functools

import jax
import jax.numpy as jnp
from jax.experimental import pallas as pl
from jax.experimental.pallas import tpu as pltpu

_CS = 512


def _fp_kernel(mask_ref, out_ref):
    n = mask_ref.shape[1]
    iota = jax.lax.broadcasted_iota(jnp.int32, mask_ref.shape, 1)
    cand = jnp.where(mask_ref[...] != 0, n, iota)
    fp = jnp.min(cand, axis=1, keepdims=True)
    out_ref[...] = jnp.broadcast_to(fp, out_ref.shape)


def _pad_copy_kernel(n, fp_ref, x_ref, start_ref, end_ref, out_ref, carry):
    bi = pl.program_id(0)
    i = pl.program_id(1)
    cs, d = x_ref.shape
    cur = x_ref[...]
    shifted = jnp.concatenate([carry[...], cur[: cs - 1, :]], axis=0)
    rows = jax.lax.broadcasted_iota(jnp.int32, (cs, 1), 0) + i * cs
    fp = fp_ref[bi]
    val = jnp.where(rows == 0, start_ref[...], shifted)
    val = jnp.where(rows == n + 1, 0.0, val)
    val = jnp.where(rows == fp, end_ref[...], val)
    out_ref[...] = val
    carry[...] = cur[cs - 1 :, :]


def kernel(x, protein_mask, start, end):
    b, n, d = x.shape
    mask_i32 = protein_mask.astype(jnp.int32)
    fp_full = pl.pallas_call(
        _fp_kernel,
        out_shape=jax.ShapeDtypeStruct((b, 128), jnp.int32),
    )(mask_i32)
    fp = fp_full[:, 0]

    cs = _CS
    nxb = n // cs
    nob = (n + 2 + cs - 1) // cs

    out = pl.pallas_call(
        functools.partial(_pad_copy_kernel, n),
        grid_spec=pltpu.PrefetchScalarGridSpec(
            num_scalar_prefetch=1,
            grid=(b, nob),
            in_specs=[
                pl.BlockSpec(
                    (None, cs, d),
                    lambda bi, i, *_: (bi, jnp.minimum(i, nxb - 1), 0),
                ),
                pl.BlockSpec((1, d), lambda bi, i, *_: (0, 0)),
                pl.BlockSpec((1, d), lambda bi, i, *_: (0, 0)),
            ],
            out_specs=pl.BlockSpec((None, cs, d), lambda bi, i, *_: (bi, i, 0)),
            scratch_shapes=[
                pltpu.VMEM((1, d), jnp.float32),
            ],
        ),
        out_shape=jax.ShapeDtypeStruct((b, n + 2, d), jnp.float32),
    )(fp, x, start.reshape(1, d), end.reshape(1, d))
    return out
